# baseline (speedup 1.0000x reference)
import functools

import jax
import jax.numpy as jnp
from jax import lax
from jax.experimental import pallas as pl
from jax.experimental.pallas import tpu as pltpu
from jax.experimental.pallas import tpu_sc as plsc

N_NODES = 10000
F_IN = 128
F_HID = 256
F_OUT = 64

NC, NS = 2, 16
NW = NC * NS
CHUNK = 128
NCOLS = 64
NBUF = 4
RPAD = 10240


def _spmm_parts(xs, src2d, dst2d, zrows, zdeg, with_deg):
    nx = len(xs)
    chunks_total = src2d.shape[0]
    ct = chunks_total // NW
    rpt = RPAD // NS

    mesh = plsc.VectorSubcoreMesh(
        core_axis_name="c", subcore_axis_name="s", num_cores=NC, num_subcores=NS
    )

    @functools.partial(
        pl.kernel,
        mesh=mesh,
        compiler_params=pltpu.CompilerParams(use_tc_tiling_on_sc=False),
        out_type=tuple(
            [jax.ShapeDtypeStruct((NC, RPAD, NCOLS), jnp.float32)] * nx
            + [jax.ShapeDtypeStruct((NC, RPAD), jnp.float32)]
        ),
        scratch_types=[
            pltpu.VMEM((ct, CHUNK), jnp.int32),
            pltpu.VMEM((ct, CHUNK), jnp.int32),
            pltpu.VMEM((NBUF * CHUNK, NCOLS), jnp.float32),
            pltpu.VMEM((CHUNK,), jnp.float32),
            pltpu.VMEM_SHARED((RPAD, NCOLS), jnp.float32),
            pltpu.VMEM_SHARED((RPAD,), jnp.float32),
            pltpu.SemaphoreType.DMA,
        ],
    )
    def k(*refs):
        x_hbms = refs[:nx]
        src_hbm, dst_hbm, zrows_hbm, zdeg_hbm = refs[nx:nx + 4]
        out_hbms = refs[nx + 4:2 * nx + 4]
        deg_hbm = refs[2 * nx + 4]
        sidx, didx, buf, ones_v, acc, dacc, sem0 = refs[2 * nx + 5:]

        cid = lax.axis_index("c")
        sid = lax.axis_index("s")
        wid = sid * NC + cid
        rbase = sid * rpt
        if with_deg:
            pltpu.sync_copy(zdeg_hbm.at[pl.ds(rbase, rpt)], dacc.at[pl.ds(rbase, rpt)])
            for i in range(CHUNK // 16):
                ones_v[pl.ds(i * 16, 16)] = jnp.ones((16,), jnp.float32)
        cbase = wid * ct
        pltpu.sync_copy(src_hbm.at[pl.ds(cbase, ct)], sidx)
        pltpu.sync_copy(dst_hbm.at[pl.ds(cbase, ct)], didx)

        def bslice(j):
            return buf.at[pl.ds((j % NBUF) * CHUNK, CHUNK)]

        for h in range(nx):
            x_hbm = x_hbms[h]
            deg_now = with_deg and h == 0
            pltpu.sync_copy(zrows_hbm.at[pl.ds(rbase, rpt)],
                            acc.at[pl.ds(rbase, rpt)])
            plsc.subcore_barrier()

            for p in range(NBUF - 1):
                pltpu.async_copy(x_hbm.at[sidx.at[p]], bslice(p), sem0)

            def body(j, _):
                pltpu.async_copy(x_hbm.at[sidx.at[j + NBUF - 1]], bslice(j + NBUF - 1), sem0)
                pltpu.make_async_copy(x_hbm.at[sidx.at[j]], bslice(j), sem0).wait()
                pltpu.sync_copy(bslice(j), acc.at[didx.at[j]], add=True)
                if deg_now:
                    pltpu.sync_copy(ones_v, dacc.at[didx.at[j]], add=True)
                return 0

            lax.fori_loop(0, ct - (NBUF - 1), body, 0)
            for p in range(NBUF - 1):
                jt = ct - (NBUF - 1) + p
                pltpu.make_async_copy(
                    x_hbm.at[sidx.at[jt]], bslice(jt), sem0).wait()
                pltpu.sync_copy(bslice(jt), acc.at[didx.at[jt]], add=True)
                if deg_now:
                    pltpu.sync_copy(ones_v, dacc.at[didx.at[jt]], add=True)

            plsc.subcore_barrier()
            pltpu.sync_copy(acc.at[pl.ds(rbase, rpt)],
                            out_hbms[h].at[cid, pl.ds(rbase, rpt)])
        if with_deg:
            pltpu.sync_copy(dacc.at[pl.ds(rbase, rpt)],
                            deg_hbm.at[cid, pl.ds(rbase, rpt)])

    res = k(*xs, src2d, dst2d, zrows, zdeg)
    return list(res[:nx]), res[nx]


def _fused_mlp(axpL, axpR, degp, W1, b1, W2):
    BLK = 512
    grid = (RPAD // BLK,)
    degp3 = degp.reshape(NC, RPAD, 1)
    b1r = b1.reshape(1, F_HID)
    W1a, W1b = W1[:NCOLS], W1[NCOLS:]

    def body(aL_ref, aR_ref, d_ref, w1a_ref, w1b_ref, b1_ref, w2_ref, o_ref):
        aL = aL_ref[0] + aL_ref[1]
        aR = aR_ref[0] + aR_ref[1]
        deg = d_ref[0] + d_ref[1]
        h = (jnp.dot(aL, w1a_ref[...], preferred_element_type=jnp.float32)
             + jnp.dot(aR, w1b_ref[...], preferred_element_type=jnp.float32))
        h = jnp.maximum(h + deg * b1_ref[...], 0.0)
        o_ref[...] = jnp.dot(h, w2_ref[...], preferred_element_type=jnp.float32)

    return pl.pallas_call(
        body,
        grid=grid,
        in_specs=[
            pl.BlockSpec((NC, BLK, NCOLS), lambda i: (0, i, 0)),
            pl.BlockSpec((NC, BLK, NCOLS), lambda i: (0, i, 0)),
            pl.BlockSpec((NC, BLK, 1), lambda i: (0, i, 0)),
            pl.BlockSpec((NCOLS, F_HID), lambda i: (0, 0)),
            pl.BlockSpec((NCOLS, F_HID), lambda i: (0, 0)),
            pl.BlockSpec((1, F_HID), lambda i: (0, 0)),
            pl.BlockSpec((F_HID, F_OUT), lambda i: (0, 0)),
        ],
        out_specs=pl.BlockSpec((BLK, F_OUT), lambda i: (i, 0)),
        out_shape=jax.ShapeDtypeStruct((RPAD, F_OUT), jnp.float32),
    )(axpL, axpR, degp3, W1a, W1b, b1r, W2)


def _combine(outp, degp, b2):
    BLK = 1024
    grid = (RPAD // BLK,)
    degp3 = degp.reshape(NC, RPAD, 1)
    b2r = b2.reshape(1, F_OUT)

    def body(o_ref, d_ref, b2_ref, out_ref):
        deg = d_ref[0] + d_ref[1]
        out_ref[...] = o_ref[0] + o_ref[1] + deg * b2_ref[...]

    return pl.pallas_call(
        body,
        grid=grid,
        in_specs=[
            pl.BlockSpec((NC, BLK, F_OUT), lambda i: (0, i, 0)),
            pl.BlockSpec((NC, BLK, 1), lambda i: (0, i, 0)),
            pl.BlockSpec((1, F_OUT), lambda i: (0, 0)),
        ],
        out_specs=pl.BlockSpec((BLK, F_OUT), lambda i: (i, 0)),
        out_shape=jax.ShapeDtypeStruct((RPAD, F_OUT), jnp.float32),
    )(outp, degp3, b2r)


def kernel(x, edge_index, W1, b1, W2, b2):
    n_edges = edge_index.shape[1]
    src = edge_index[0].astype(jnp.int32)
    dst = edge_index[1].astype(jnp.int32)

    gran = NW * CHUNK * 8
    epad = -(-n_edges // gran) * gran
    src2d = jnp.concatenate(
        [src, jnp.zeros((epad - n_edges,), jnp.int32)]).reshape(-1, CHUNK)
    dst2d = jnp.concatenate(
        [dst, jnp.full((epad - n_edges,), N_NODES, jnp.int32)]).reshape(-1, CHUNK)

    zrows = jnp.zeros((RPAD, NCOLS), jnp.float32)
    zdeg = jnp.zeros((RPAD,), jnp.float32)

    xL = jnp.asarray(x[:, :NCOLS], jnp.float32)
    xR = jnp.asarray(x[:, NCOLS:], jnp.float32)

    (axpL, axpR), degp = _spmm_parts(
        [xL, xR], src2d, dst2d, zrows, zdeg, with_deg=True)
    s2 = _fused_mlp(axpL, axpR, degp, W1, b1, W2)
    (outp,), _ = _spmm_parts([s2], src2d, dst2d, zrows, zdeg, with_deg=False)
    out = _combine(outp, degp, b2)
    return out[:N_NODES]


# speedup vs baseline: 1.0304x; 1.0304x over previous
import functools

import jax
import jax.numpy as jnp
from jax import lax
from jax.experimental import pallas as pl
from jax.experimental.pallas import tpu as pltpu
from jax.experimental.pallas import tpu_sc as plsc

N_NODES = 10000
F_IN = 128
F_HID = 256
F_OUT = 64

NC, NS = 2, 16
NW = NC * NS
CHUNK = 128
CT0_NUM, CT1_NUM = 7, 3
RPAD = 10240


def _spmm_parts(x, sd3d, zrows, zdeg, ncols, with_deg):
    chunks_total = sd3d.shape[0]
    ct = chunks_total // NW
    ct0 = (2 * ct * CT0_NUM // (CT0_NUM + CT1_NUM)) // 8 * 8
    ct1 = 2 * ct - ct0
    ctm = max(ct0, ct1)
    rpt = RPAD // NS

    mesh = plsc.VectorSubcoreMesh(
        core_axis_name="c", subcore_axis_name="s", num_cores=NC, num_subcores=NS
    )

    @functools.partial(
        pl.kernel,
        mesh=mesh,
        compiler_params=pltpu.CompilerParams(use_tc_tiling_on_sc=False),
        out_type=(
            jax.ShapeDtypeStruct((NC, RPAD, ncols), jnp.float32),
            jax.ShapeDtypeStruct((NC, RPAD), jnp.float32),
        ),
        scratch_types=[
            pltpu.VMEM((4, 2, CHUNK), jnp.int32),
            pltpu.VMEM((2 * CHUNK, ncols), jnp.float32),
            pltpu.VMEM((CHUNK,), jnp.float32),
            pltpu.VMEM_SHARED((RPAD, ncols), jnp.float32),
            pltpu.VMEM_SHARED((RPAD,), jnp.float32),
            pltpu.SemaphoreType.DMA,
            pltpu.SemaphoreType.DMA,
        ],
    )
    def k(x_hbm, sd_hbm, zrows_hbm, zdeg_hbm, out_hbm, deg_hbm,
          idxb, buf, ones_v, acc, dacc, sem0, semi):
        cid = lax.axis_index("c")
        sid = lax.axis_index("s")
        rbase = sid * rpt
        ctc = jnp.where(cid == 0, ct0, ct1)
        if with_deg:
            pltpu.sync_copy(zdeg_hbm.at[pl.ds(rbase, rpt)],
                            dacc.at[pl.ds(rbase, rpt)])
            for i in range(CHUNK // 16):
                ones_v[pl.ds(i * 16, 16)] = jnp.ones((16,), jnp.float32)
        cbase = jnp.where(cid == 0, sid * ct0, NS * ct0 + sid * ct1)
        pltpu.sync_copy(zrows_hbm.at[pl.ds(rbase, rpt)], acc.at[pl.ds(rbase, rpt)])
        plsc.subcore_barrier()

        def bslice(j):
            return buf.at[pl.ds((j % 2) * CHUNK, CHUNK)]

        def islot(j):
            return idxb.at[j % 4]

        def idx_copy(j):
            pltpu.async_copy(sd_hbm.at[cbase + j], islot(j), semi)

        def wait_idx(j):
            pltpu.make_async_copy(sd_hbm.at[cbase + j], islot(j), semi).wait()

        def gather(j):
            pltpu.async_copy(x_hbm.at[idxb.at[j % 4, 0]], bslice(j), sem0)

        idx_copy(0)
        idx_copy(1)
        wait_idx(0)
        gather(0)

        def body(j, _):
            @pl.when(j + 2 < ctc)
            def _():
                idx_copy(j + 2)

            @pl.when(j + 1 < ctc)
            def _():
                wait_idx(j + 1)
                gather(j + 1)

            pltpu.make_async_copy(x_hbm.at[idxb.at[j % 4, 0]], bslice(j), sem0).wait()
            pltpu.sync_copy(bslice(j), acc.at[idxb.at[j % 4, 1]], add=True)
            if with_deg:
                pltpu.sync_copy(ones_v, dacc.at[idxb.at[j % 4, 1]], add=True)
            return 0

        lax.fori_loop(0, ctc, body, 0)
        plsc.subcore_barrier()
        pltpu.sync_copy(acc.at[pl.ds(rbase, rpt)],
                        out_hbm.at[cid, pl.ds(rbase, rpt)])
        if with_deg:
            pltpu.sync_copy(dacc.at[pl.ds(rbase, rpt)],
                            deg_hbm.at[cid, pl.ds(rbase, rpt)])

    return k(x, sd3d, zrows, zdeg)


def _fused_mlp(axp, degp, W1, b1, W2):
    BLK = 512
    grid = (RPAD // BLK,)
    degp3 = degp.reshape(NC, RPAD, 1)
    b1r = b1.reshape(1, F_HID)

    def body(a_ref, d_ref, w1_ref, b1_ref, w2_ref, o_ref):
        a = a_ref[0] + a_ref[1]
        deg = d_ref[0] + d_ref[1]
        h = jnp.dot(a, w1_ref[...], preferred_element_type=jnp.float32)
        h = jnp.maximum(h + deg * b1_ref[...], 0.0)
        o_ref[...] = jnp.dot(h, w2_ref[...], preferred_element_type=jnp.float32)

    return pl.pallas_call(
        body,
        grid=grid,
        in_specs=[
            pl.BlockSpec((NC, BLK, F_IN), lambda i: (0, i, 0)),
            pl.BlockSpec((NC, BLK, 1), lambda i: (0, i, 0)),
            pl.BlockSpec((F_IN, F_HID), lambda i: (0, 0)),
            pl.BlockSpec((1, F_HID), lambda i: (0, 0)),
            pl.BlockSpec((F_HID, F_OUT), lambda i: (0, 0)),
        ],
        out_specs=pl.BlockSpec((BLK, F_OUT), lambda i: (i, 0)),
        out_shape=jax.ShapeDtypeStruct((RPAD, F_OUT), jnp.float32),
    )(axp, degp3, W1, b1r, W2)


def _combine(outp, degp, b2):
    BLK = 1024
    grid = (RPAD // BLK,)
    degp3 = degp.reshape(NC, RPAD, 1)
    b2r = b2.reshape(1, F_OUT)

    def body(o_ref, d_ref, b2_ref, out_ref):
        deg = d_ref[0] + d_ref[1]
        out_ref[...] = o_ref[0] + o_ref[1] + deg * b2_ref[...]

    return pl.pallas_call(
        body,
        grid=grid,
        in_specs=[
            pl.BlockSpec((NC, BLK, F_OUT), lambda i: (0, i, 0)),
            pl.BlockSpec((NC, BLK, 1), lambda i: (0, i, 0)),
            pl.BlockSpec((1, F_OUT), lambda i: (0, 0)),
        ],
        out_specs=pl.BlockSpec((BLK, F_OUT), lambda i: (i, 0)),
        out_shape=jax.ShapeDtypeStruct((RPAD, F_OUT), jnp.float32),
    )(outp, degp3, b2r)


def kernel(x, edge_index, W1, b1, W2, b2):
    n_edges = edge_index.shape[1]
    src = edge_index[0].astype(jnp.int32)
    dst = edge_index[1].astype(jnp.int32)

    gran = NW * CHUNK * 8
    epad = -(-n_edges // gran) * gran
    src2d = jnp.concatenate(
        [src, jnp.zeros((epad - n_edges,), jnp.int32)]).reshape(-1, 1, CHUNK)
    dst2d = jnp.concatenate(
        [dst, jnp.full((epad - n_edges,), N_NODES, jnp.int32)]).reshape(-1, 1, CHUNK)
    sd3d = jnp.concatenate([src2d, dst2d], axis=1)

    zrows = jnp.zeros((RPAD, F_IN), jnp.float32)
    zdeg = jnp.zeros((RPAD,), jnp.float32)

    axp, degp = _spmm_parts(x, sd3d, zrows, zdeg, F_IN, with_deg=True)
    s2 = _fused_mlp(axp, degp, W1, b1, W2)
    outp, _ = _spmm_parts(s2, sd3d, zrows[:, :F_OUT], zdeg, F_OUT, with_deg=False)
    out = _combine(outp, degp, b2)
    return out[:N_NODES]


# speedup vs baseline: 1.2310x; 1.1946x over previous
import functools

import jax
import jax.numpy as jnp
from jax import lax
from jax.experimental import pallas as pl
from jax.experimental.pallas import tpu as pltpu
from jax.experimental.pallas import tpu_sc as plsc

N_NODES = 10000
F_IN = 128
F_HID = 256
F_OUT = 64

NC, NS = 2, 16
NW = NC * NS
CHUNK = 128
CT0_NUM, CT1_NUM = 7, 3
RPAD = 10240


def _spmm_parts(x, sd3d, zrows, zdeg, ncols, with_deg, cache_src=False):
    xr = x.shape[0]
    chunks_total = sd3d.shape[0]
    ct = chunks_total // NW
    ct0 = (2 * ct * CT0_NUM // (CT0_NUM + CT1_NUM)) // 8 * 8
    ct1 = 2 * ct - ct0
    ctm = max(ct0, ct1)
    rpt = RPAD // NS

    mesh = plsc.VectorSubcoreMesh(
        core_axis_name="c", subcore_axis_name="s", num_cores=NC, num_subcores=NS
    )

    @functools.partial(
        pl.kernel,
        mesh=mesh,
        compiler_params=pltpu.CompilerParams(use_tc_tiling_on_sc=False),
        out_type=(
            jax.ShapeDtypeStruct((NC, RPAD, ncols), jnp.float32),
            jax.ShapeDtypeStruct((NC, RPAD), jnp.float32),
        ),
        scratch_types=[
            pltpu.VMEM((4, 2, CHUNK), jnp.int32),
            pltpu.VMEM((2 * CHUNK, ncols), jnp.float32),
            pltpu.VMEM((CHUNK,), jnp.float32),
            pltpu.VMEM_SHARED((RPAD, ncols), jnp.float32),
            pltpu.VMEM_SHARED((RPAD,), jnp.float32),
            pltpu.VMEM_SHARED((xr if cache_src else 8, ncols), jnp.float32),
            pltpu.SemaphoreType.DMA,
            pltpu.SemaphoreType.DMA,
        ],
    )
    def k(x_hbm, sd_hbm, zrows_hbm, zdeg_hbm, out_hbm, deg_hbm,
          idxb, buf, ones_v, acc, dacc, xcache, sem0, semi):
        cid = lax.axis_index("c")
        sid = lax.axis_index("s")
        rbase = sid * rpt
        ctc = jnp.where(cid == 0, ct0, ct1)
        if with_deg:
            pltpu.sync_copy(zdeg_hbm.at[pl.ds(rbase, rpt)],
                            dacc.at[pl.ds(rbase, rpt)])
            for i in range(CHUNK // 16):
                ones_v[pl.ds(i * 16, 16)] = jnp.ones((16,), jnp.float32)
        cbase = jnp.where(cid == 0, sid * ct0, NS * ct0 + sid * ct1)
        if cache_src:
            xpt = xr // NS
            pltpu.sync_copy(x_hbm.at[pl.ds(sid * xpt, xpt)],
                            xcache.at[pl.ds(sid * xpt, xpt)])
        pltpu.sync_copy(zrows_hbm.at[pl.ds(rbase, rpt)], acc.at[pl.ds(rbase, rpt)])
        plsc.subcore_barrier()

        def bslice(j):
            return buf.at[pl.ds((j % 2) * CHUNK, CHUNK)]

        def islot(j):
            return idxb.at[j % 4]

        def idx_copy(j):
            pltpu.async_copy(sd_hbm.at[cbase + j], islot(j), semi)

        def wait_idx(j):
            pltpu.make_async_copy(sd_hbm.at[cbase + j], islot(j), semi).wait()

        gsrc = xcache if cache_src else x_hbm

        def gather(j):
            pltpu.async_copy(gsrc.at[idxb.at[j % 4, 0]], bslice(j), sem0)

        idx_copy(0)
        idx_copy(1)
        wait_idx(0)
        gather(0)

        def body(j, _):
            @pl.when(j + 2 < ctc)
            def _():
                idx_copy(j + 2)

            @pl.when(j + 1 < ctc)
            def _():
                wait_idx(j + 1)
                gather(j + 1)

            pltpu.make_async_copy(gsrc.at[idxb.at[j % 4, 0]], bslice(j), sem0).wait()
            pltpu.sync_copy(bslice(j), acc.at[idxb.at[j % 4, 1]], add=True)
            if with_deg:
                pltpu.sync_copy(ones_v, dacc.at[idxb.at[j % 4, 1]], add=True)
            return 0

        lax.fori_loop(0, ctc, body, 0)
        plsc.subcore_barrier()
        pltpu.sync_copy(acc.at[pl.ds(rbase, rpt)],
                        out_hbm.at[cid, pl.ds(rbase, rpt)])
        if with_deg:
            pltpu.sync_copy(dacc.at[pl.ds(rbase, rpt)],
                            deg_hbm.at[cid, pl.ds(rbase, rpt)])

    return k(x, sd3d, zrows, zdeg)


def _fused_mlp(axp, degp, W1, b1, W2):
    BLK = 512
    grid = (RPAD // BLK,)
    degp3 = degp.reshape(NC, RPAD, 1)
    b1r = b1.reshape(1, F_HID)

    def body(a_ref, d_ref, w1_ref, b1_ref, w2_ref, o_ref):
        a = a_ref[0] + a_ref[1]
        deg = d_ref[0] + d_ref[1]
        h = jnp.dot(a, w1_ref[...], preferred_element_type=jnp.float32)
        h = jnp.maximum(h + deg * b1_ref[...], 0.0)
        o_ref[...] = jnp.dot(h, w2_ref[...], preferred_element_type=jnp.float32)

    return pl.pallas_call(
        body,
        grid=grid,
        in_specs=[
            pl.BlockSpec((NC, BLK, F_IN), lambda i: (0, i, 0)),
            pl.BlockSpec((NC, BLK, 1), lambda i: (0, i, 0)),
            pl.BlockSpec((F_IN, F_HID), lambda i: (0, 0)),
            pl.BlockSpec((1, F_HID), lambda i: (0, 0)),
            pl.BlockSpec((F_HID, F_OUT), lambda i: (0, 0)),
        ],
        out_specs=pl.BlockSpec((BLK, F_OUT), lambda i: (i, 0)),
        out_shape=jax.ShapeDtypeStruct((RPAD, F_OUT), jnp.float32),
    )(axp, degp3, W1, b1r, W2)


def _combine(outp, degp, b2):
    BLK = 1024
    grid = (RPAD // BLK,)
    degp3 = degp.reshape(NC, RPAD, 1)
    b2r = b2.reshape(1, F_OUT)

    def body(o_ref, d_ref, b2_ref, out_ref):
        deg = d_ref[0] + d_ref[1]
        out_ref[...] = o_ref[0] + o_ref[1] + deg * b2_ref[...]

    return pl.pallas_call(
        body,
        grid=grid,
        in_specs=[
            pl.BlockSpec((NC, BLK, F_OUT), lambda i: (0, i, 0)),
            pl.BlockSpec((NC, BLK, 1), lambda i: (0, i, 0)),
            pl.BlockSpec((1, F_OUT), lambda i: (0, 0)),
        ],
        out_specs=pl.BlockSpec((BLK, F_OUT), lambda i: (i, 0)),
        out_shape=jax.ShapeDtypeStruct((RPAD, F_OUT), jnp.float32),
    )(outp, degp3, b2r)


def kernel(x, edge_index, W1, b1, W2, b2):
    n_edges = edge_index.shape[1]
    src = edge_index[0].astype(jnp.int32)
    dst = edge_index[1].astype(jnp.int32)

    gran = NW * CHUNK * 8
    epad = -(-n_edges // gran) * gran
    src2d = jnp.concatenate(
        [src, jnp.zeros((epad - n_edges,), jnp.int32)]).reshape(-1, 1, CHUNK)
    dst2d = jnp.concatenate(
        [dst, jnp.full((epad - n_edges,), N_NODES, jnp.int32)]).reshape(-1, 1, CHUNK)
    sd3d = jnp.concatenate([src2d, dst2d], axis=1)

    zrows = jnp.zeros((RPAD, F_IN), jnp.float32)
    zdeg = jnp.zeros((RPAD,), jnp.float32)

    axp, degp = _spmm_parts(x, sd3d, zrows, zdeg, F_IN, with_deg=True)
    s2 = _fused_mlp(axp, degp, W1, b1, W2)
    outp, _ = _spmm_parts(s2, sd3d, zrows[:, :F_OUT], zdeg, F_OUT,
                          with_deg=False, cache_src=True)
    out = _combine(outp, degp, b2)
    return out[:N_NODES]


# speedup vs baseline: 1.7058x; 1.3857x over previous
import functools

import jax
import jax.numpy as jnp
from jax import lax
from jax.experimental import pallas as pl
from jax.experimental.pallas import tpu as pltpu
from jax.experimental.pallas import tpu_sc as plsc

N_NODES = 10000
F_IN = 128
F_HID = 256
F_OUT = 64

NC, NS = 2, 16
NW = NC * NS
CHUNK = 128
NCOL2 = 64
CT0_NUM, CT1_NUM = 7, 3
RPAD = 10240


def _spmm_parts(xs, sd3d, zrows, zdeg, with_deg):
    nx = len(xs)
    xr = xs[0].shape[0]
    ncols = xs[0].shape[1]
    chunks_total = sd3d.shape[0]
    ct = chunks_total // NW
    ct0 = (2 * ct * CT0_NUM // (CT0_NUM + CT1_NUM)) // 8 * 8
    ct1 = 2 * ct - ct0
    ctm = max(ct0, ct1)
    rpt = RPAD // NS

    mesh = plsc.VectorSubcoreMesh(
        core_axis_name="c", subcore_axis_name="s", num_cores=NC, num_subcores=NS
    )

    @functools.partial(
        pl.kernel,
        mesh=mesh,
        compiler_params=pltpu.CompilerParams(use_tc_tiling_on_sc=False),
        out_type=tuple(
            [jax.ShapeDtypeStruct((NC, RPAD, ncols), jnp.float32)] * nx
            + [jax.ShapeDtypeStruct((NC, RPAD), jnp.float32)]
        ),
        scratch_types=[
            pltpu.VMEM((4, 2, CHUNK), jnp.int32),
            pltpu.VMEM((2 * CHUNK, ncols), jnp.float32),
            pltpu.VMEM((CHUNK,), jnp.float32),
            pltpu.VMEM_SHARED((RPAD, ncols), jnp.float32),
            pltpu.VMEM_SHARED((RPAD,), jnp.float32),
            pltpu.VMEM_SHARED((xr, ncols), jnp.float32),
            pltpu.SemaphoreType.DMA,
            pltpu.SemaphoreType.DMA,
        ],
    )
    def k(*refs):
        x_hbms = refs[:nx]
        sd_hbm, zrows_hbm, zdeg_hbm = refs[nx:nx + 3]
        out_hbms = refs[nx + 3:2 * nx + 3]
        deg_hbm = refs[2 * nx + 3]
        idxb, buf, ones_v, acc, dacc, xcache, sem0, semi = refs[2 * nx + 4:]

        cid = lax.axis_index("c")
        sid = lax.axis_index("s")
        rbase = sid * rpt
        xpt = xr // NS
        ctc = jnp.where(cid == 0, ct0, ct1)
        cbase = jnp.where(cid == 0, sid * ct0, NS * ct0 + sid * ct1)
        if with_deg:
            pltpu.sync_copy(zdeg_hbm.at[pl.ds(rbase, rpt)],
                            dacc.at[pl.ds(rbase, rpt)])
            for i in range(CHUNK // 16):
                ones_v[pl.ds(i * 16, 16)] = jnp.ones((16,), jnp.float32)

        def bslice(j):
            return buf.at[pl.ds((j % 2) * CHUNK, CHUNK)]

        def islot(j):
            return idxb.at[j % 4]

        def idx_copy(j):
            pltpu.async_copy(sd_hbm.at[cbase + j], islot(j), semi)

        def wait_idx(j):
            pltpu.make_async_copy(sd_hbm.at[cbase + j], islot(j), semi).wait()

        def gather(j):
            pltpu.async_copy(xcache.at[idxb.at[j % 4, 0]], bslice(j), sem0)

        for h in range(nx):
            deg_now = with_deg and h == 0
            pltpu.sync_copy(x_hbms[h].at[pl.ds(sid * xpt, xpt)],
                            xcache.at[pl.ds(sid * xpt, xpt)])
            pltpu.sync_copy(zrows_hbm.at[pl.ds(rbase, rpt)],
                            acc.at[pl.ds(rbase, rpt)])
            plsc.subcore_barrier()

            idx_copy(0)
            idx_copy(1)
            wait_idx(0)
            gather(0)

            def body(j, _):
                @pl.when(j + 2 < ctc)
                def _():
                    idx_copy(j + 2)

                @pl.when(j + 1 < ctc)
                def _():
                    wait_idx(j + 1)
                    gather(j + 1)

                pltpu.make_async_copy(
                    xcache.at[idxb.at[j % 4, 0]], bslice(j), sem0).wait()
                pltpu.sync_copy(bslice(j), acc.at[idxb.at[j % 4, 1]], add=True)
                if deg_now:
                    pltpu.sync_copy(ones_v, dacc.at[idxb.at[j % 4, 1]], add=True)
                return 0

            lax.fori_loop(0, ctc, body, 0)
            plsc.subcore_barrier()
            pltpu.sync_copy(acc.at[pl.ds(rbase, rpt)],
                            out_hbms[h].at[cid, pl.ds(rbase, rpt)])
        if with_deg:
            pltpu.sync_copy(dacc.at[pl.ds(rbase, rpt)],
                            deg_hbm.at[cid, pl.ds(rbase, rpt)])

    res = k(*xs, sd3d, zrows, zdeg)
    return list(res[:nx]), res[nx]


def _fused_mlp(axpL, axpR, degp, W1, b1, W2):
    BLK = 512
    grid = (RPAD // BLK,)
    degp3 = degp.reshape(NC, RPAD, 1)
    b1r = b1.reshape(1, F_HID)
    W1a, W1b = W1[:NCOL2], W1[NCOL2:]

    def body(aL_ref, aR_ref, d_ref, w1a_ref, w1b_ref, b1_ref, w2_ref, o_ref):
        aL = aL_ref[0] + aL_ref[1]
        aR = aR_ref[0] + aR_ref[1]
        deg = d_ref[0] + d_ref[1]
        h = (jnp.dot(aL, w1a_ref[...], preferred_element_type=jnp.float32)
             + jnp.dot(aR, w1b_ref[...], preferred_element_type=jnp.float32))
        h = jnp.maximum(h + deg * b1_ref[...], 0.0)
        o_ref[...] = jnp.dot(h, w2_ref[...], preferred_element_type=jnp.float32)

    return pl.pallas_call(
        body,
        grid=grid,
        in_specs=[
            pl.BlockSpec((NC, BLK, NCOL2), lambda i: (0, i, 0)),
            pl.BlockSpec((NC, BLK, NCOL2), lambda i: (0, i, 0)),
            pl.BlockSpec((NC, BLK, 1), lambda i: (0, i, 0)),
            pl.BlockSpec((NCOL2, F_HID), lambda i: (0, 0)),
            pl.BlockSpec((NCOL2, F_HID), lambda i: (0, 0)),
            pl.BlockSpec((1, F_HID), lambda i: (0, 0)),
            pl.BlockSpec((F_HID, F_OUT), lambda i: (0, 0)),
        ],
        out_specs=pl.BlockSpec((BLK, F_OUT), lambda i: (i, 0)),
        out_shape=jax.ShapeDtypeStruct((RPAD, F_OUT), jnp.float32),
    )(axpL, axpR, degp3, W1a, W1b, b1r, W2)


def _combine(outp, degp, b2):
    BLK = 1024
    grid = (RPAD // BLK,)
    degp3 = degp.reshape(NC, RPAD, 1)
    b2r = b2.reshape(1, F_OUT)

    def body(o_ref, d_ref, b2_ref, out_ref):
        deg = d_ref[0] + d_ref[1]
        out_ref[...] = o_ref[0] + o_ref[1] + deg * b2_ref[...]

    return pl.pallas_call(
        body,
        grid=grid,
        in_specs=[
            pl.BlockSpec((NC, BLK, F_OUT), lambda i: (0, i, 0)),
            pl.BlockSpec((NC, BLK, 1), lambda i: (0, i, 0)),
            pl.BlockSpec((1, F_OUT), lambda i: (0, 0)),
        ],
        out_specs=pl.BlockSpec((BLK, F_OUT), lambda i: (i, 0)),
        out_shape=jax.ShapeDtypeStruct((RPAD, F_OUT), jnp.float32),
    )(outp, degp3, b2r)


def kernel(x, edge_index, W1, b1, W2, b2):
    n_edges = edge_index.shape[1]
    src = edge_index[0].astype(jnp.int32)
    dst = edge_index[1].astype(jnp.int32)

    gran = NW * CHUNK * 8
    epad = -(-n_edges // gran) * gran
    src2d = jnp.concatenate(
        [src, jnp.zeros((epad - n_edges,), jnp.int32)]).reshape(-1, 1, CHUNK)
    dst2d = jnp.concatenate(
        [dst, jnp.full((epad - n_edges,), N_NODES, jnp.int32)]).reshape(-1, 1, CHUNK)
    sd3d = jnp.concatenate([src2d, dst2d], axis=1)

    zrows = jnp.zeros((RPAD, F_OUT), jnp.float32)
    zdeg = jnp.zeros((RPAD,), jnp.float32)

    XR = 10112
    xpad = jnp.zeros((XR - N_NODES, NCOL2), jnp.float32)
    xL = jnp.concatenate([x[:, :NCOL2], xpad])
    xR = jnp.concatenate([x[:, NCOL2:], xpad])

    (axpL, axpR), degp = _spmm_parts([xL, xR], sd3d, zrows, zdeg, with_deg=True)
    s2 = _fused_mlp(axpL, axpR, degp, W1, b1, W2)
    (outp,), _ = _spmm_parts([s2], sd3d, zrows, zdeg, with_deg=False)
    out = _combine(outp, degp, b2)
    return out[:N_NODES]


# speedup vs baseline: 2.0550x; 1.2047x over previous
import functools

import jax
import jax.numpy as jnp
from jax import lax
from jax.experimental import pallas as pl
from jax.experimental.pallas import tpu as pltpu
from jax.experimental.pallas import tpu_sc as plsc

N_NODES = 10000
F_IN = 128
F_HID = 256
F_OUT = 64

NC, NS = 2, 16
NW = NC * NS
CHUNK = 128
NCOL2 = 64
CT0_NUM, CT1_NUM = 1, 1
RPAD = 10240


def _spmm_parts(xs, sd3d, zrows, zdeg, with_deg):
    nx = len(xs)
    xr = xs[0].shape[0]
    ncols = xs[0].shape[1]
    chunks_total = sd3d.shape[0]
    ct = chunks_total // NW
    ct0 = (2 * ct * CT0_NUM // (CT0_NUM + CT1_NUM)) // 8 * 8
    ct1 = 2 * ct - ct0
    ctm = max(ct0, ct1)
    rpt = RPAD // NS

    mesh = plsc.VectorSubcoreMesh(
        core_axis_name="c", subcore_axis_name="s", num_cores=NC, num_subcores=NS
    )

    @functools.partial(
        pl.kernel,
        mesh=mesh,
        compiler_params=pltpu.CompilerParams(use_tc_tiling_on_sc=False),
        out_type=tuple(
            [jax.ShapeDtypeStruct((NC, RPAD, ncols), jnp.float32)] * nx
            + [jax.ShapeDtypeStruct((NC, RPAD), jnp.float32)]
        ),
        scratch_types=[
            pltpu.VMEM((4, 2, CHUNK), jnp.int32),
            pltpu.VMEM((2 * CHUNK, ncols), jnp.float32),
            pltpu.VMEM((CHUNK,), jnp.float32),
            pltpu.VMEM_SHARED((RPAD, ncols), jnp.float32),
            pltpu.VMEM_SHARED((RPAD,), jnp.float32),
            pltpu.VMEM_SHARED((xr, ncols), jnp.float32),
            pltpu.SemaphoreType.DMA,
            pltpu.SemaphoreType.DMA,
        ],
    )
    def k(*refs):
        x_hbms = refs[:nx]
        sd_hbm, zrows_hbm, zdeg_hbm = refs[nx:nx + 3]
        out_hbms = refs[nx + 3:2 * nx + 3]
        deg_hbm = refs[2 * nx + 3]
        idxb, buf, ones_v, acc, dacc, xcache, sem0, semi = refs[2 * nx + 4:]

        cid = lax.axis_index("c")
        sid = lax.axis_index("s")
        rbase = sid * rpt
        xpt = xr // NS
        ctc = jnp.where(cid == 0, ct0, ct1)
        cbase = jnp.where(cid == 0, sid * ct0, NS * ct0 + sid * ct1)
        if with_deg:
            pltpu.sync_copy(zdeg_hbm.at[pl.ds(rbase, rpt)],
                            dacc.at[pl.ds(rbase, rpt)])
            for i in range(CHUNK // 16):
                ones_v[pl.ds(i * 16, 16)] = jnp.ones((16,), jnp.float32)

        def bslice(j):
            return buf.at[pl.ds((j % 2) * CHUNK, CHUNK)]

        def islot(j):
            return idxb.at[j % 4]

        def idx_copy(j):
            pltpu.async_copy(sd_hbm.at[cbase + j], islot(j), semi)

        def wait_idx(j):
            pltpu.make_async_copy(sd_hbm.at[cbase + j], islot(j), semi).wait()

        def gather(j):
            pltpu.async_copy(xcache.at[idxb.at[j % 4, 0]], bslice(j), sem0)

        for h in range(nx):
            deg_now = with_deg and h == 0
            pltpu.sync_copy(x_hbms[h].at[pl.ds(sid * xpt, xpt)],
                            xcache.at[pl.ds(sid * xpt, xpt)])
            pltpu.sync_copy(zrows_hbm.at[pl.ds(rbase, rpt)],
                            acc.at[pl.ds(rbase, rpt)])
            plsc.subcore_barrier()

            idx_copy(0)
            idx_copy(1)
            wait_idx(0)
            gather(0)

            def body(j, _):
                @pl.when(j + 2 < ctc)
                def _():
                    idx_copy(j + 2)

                @pl.when(j + 1 < ctc)
                def _():
                    wait_idx(j + 1)
                    gather(j + 1)

                pltpu.make_async_copy(
                    xcache.at[idxb.at[j % 4, 0]], bslice(j), sem0).wait()
                pltpu.sync_copy(bslice(j), acc.at[idxb.at[j % 4, 1]], add=True)
                if deg_now:
                    pltpu.sync_copy(ones_v, dacc.at[idxb.at[j % 4, 1]], add=True)
                return 0

            lax.fori_loop(0, ctc, body, 0)
            plsc.subcore_barrier()
            pltpu.sync_copy(acc.at[pl.ds(rbase, rpt)],
                            out_hbms[h].at[cid, pl.ds(rbase, rpt)])
        if with_deg:
            pltpu.sync_copy(dacc.at[pl.ds(rbase, rpt)],
                            deg_hbm.at[cid, pl.ds(rbase, rpt)])

    res = k(*xs, sd3d, zrows, zdeg)
    return list(res[:nx]), res[nx]


def _fused_mlp(axpL, axpR, degp, W1, b1, W2):
    BLK = 512
    grid = (RPAD // BLK,)
    degp3 = degp.reshape(NC, RPAD, 1)
    b1r = b1.reshape(1, F_HID)
    W1a, W1b = W1[:NCOL2], W1[NCOL2:]

    def body(aL_ref, aR_ref, d_ref, w1a_ref, w1b_ref, b1_ref, w2_ref, o_ref):
        aL = aL_ref[0] + aL_ref[1]
        aR = aR_ref[0] + aR_ref[1]
        deg = d_ref[0] + d_ref[1]
        h = (jnp.dot(aL, w1a_ref[...], preferred_element_type=jnp.float32)
             + jnp.dot(aR, w1b_ref[...], preferred_element_type=jnp.float32))
        h = jnp.maximum(h + deg * b1_ref[...], 0.0)
        o_ref[...] = jnp.dot(h, w2_ref[...], preferred_element_type=jnp.float32)

    return pl.pallas_call(
        body,
        grid=grid,
        in_specs=[
            pl.BlockSpec((NC, BLK, NCOL2), lambda i: (0, i, 0)),
            pl.BlockSpec((NC, BLK, NCOL2), lambda i: (0, i, 0)),
            pl.BlockSpec((NC, BLK, 1), lambda i: (0, i, 0)),
            pl.BlockSpec((NCOL2, F_HID), lambda i: (0, 0)),
            pl.BlockSpec((NCOL2, F_HID), lambda i: (0, 0)),
            pl.BlockSpec((1, F_HID), lambda i: (0, 0)),
            pl.BlockSpec((F_HID, F_OUT), lambda i: (0, 0)),
        ],
        out_specs=pl.BlockSpec((BLK, F_OUT), lambda i: (i, 0)),
        out_shape=jax.ShapeDtypeStruct((RPAD, F_OUT), jnp.float32),
    )(axpL, axpR, degp3, W1a, W1b, b1r, W2)


def _combine(outp, degp, b2):
    BLK = 1024
    grid = (RPAD // BLK,)
    degp3 = degp.reshape(NC, RPAD, 1)
    b2r = b2.reshape(1, F_OUT)

    def body(o_ref, d_ref, b2_ref, out_ref):
        deg = d_ref[0] + d_ref[1]
        out_ref[...] = o_ref[0] + o_ref[1] + deg * b2_ref[...]

    return pl.pallas_call(
        body,
        grid=grid,
        in_specs=[
            pl.BlockSpec((NC, BLK, F_OUT), lambda i: (0, i, 0)),
            pl.BlockSpec((NC, BLK, 1), lambda i: (0, i, 0)),
            pl.BlockSpec((1, F_OUT), lambda i: (0, 0)),
        ],
        out_specs=pl.BlockSpec((BLK, F_OUT), lambda i: (i, 0)),
        out_shape=jax.ShapeDtypeStruct((RPAD, F_OUT), jnp.float32),
    )(outp, degp3, b2r)


def kernel(x, edge_index, W1, b1, W2, b2):
    n_edges = edge_index.shape[1]
    src = edge_index[0].astype(jnp.int32)
    dst = edge_index[1].astype(jnp.int32)

    gran = NW * CHUNK * 8
    epad = -(-n_edges // gran) * gran
    src2d = jnp.concatenate(
        [src, jnp.zeros((epad - n_edges,), jnp.int32)]).reshape(-1, 1, CHUNK)
    dst2d = jnp.concatenate(
        [dst, jnp.full((epad - n_edges,), N_NODES, jnp.int32)]).reshape(-1, 1, CHUNK)
    sd3d = jnp.concatenate([src2d, dst2d], axis=1)

    zrows = jnp.zeros((RPAD, F_OUT), jnp.float32)
    zdeg = jnp.zeros((RPAD,), jnp.float32)

    XR = 10112
    xpad = jnp.zeros((XR - N_NODES, NCOL2), jnp.float32)
    xL = jnp.concatenate([x[:, :NCOL2], xpad])
    xR = jnp.concatenate([x[:, NCOL2:], xpad])

    (axpL, axpR), degp = _spmm_parts([xL, xR], sd3d, zrows, zdeg, with_deg=True)
    s2 = _fused_mlp(axpL, axpR, degp, W1, b1, W2)
    (outp,), _ = _spmm_parts([s2], sd3d, zrows, zdeg, with_deg=False)
    out = _combine(outp, degp, b2)
    return out[:N_NODES]
